# baseline (speedup 1.0000x reference)
import functools

import jax
import jax.numpy as jnp
from jax import lax
from jax.experimental import pallas as pl
from jax.experimental.pallas import tpu as pltpu
from jax.experimental.pallas import tpu_sc as plsc


def _gelu(x):
    return 0.5 * x * (1.0 + jax.lax.erf(x * 0.7071067811865476))


def _dot_t(x, w):
    return jax.lax.dot_general(
        x, w, dimension_numbers=(((1,), (1,)), ((), ())),
        preferred_element_type=jnp.float32)


_N_SUB = 2


def _fused_kernel(fm_ref, ft_ref, fi_ref, ri_ref,
                  l1w_ref, l1b_ref, l2w_ref, l2b_ref, temb_ref,
                  w1_ref, b1_ref, w2_ref, b2_ref,
                  out_ref, lg0_ref, lg1_ref, lg2_ref, w1b_ref, w2b_ref):
    @pl.when(pl.program_id(0) == 0)
    def _cast_weights():
        w1b_ref[...] = w1_ref[...].astype(jnp.bfloat16)
        w2b_ref[...] = w2_ref[...].astype(jnp.bfloat16)

    sub = out_ref.shape[0] // _N_SUB
    for k in range(_N_SUB):
        s = pl.ds(k * sub, sub)
        x = fm_ref[s, :].astype(jnp.bfloat16)
        h = _gelu(_dot_t(x, w1b_ref[...]) + b1_ref[...])
        fm = _dot_t(h.astype(jnp.bfloat16), w2b_ref[...]) + b2_ref[...]

        r = (_gelu(_dot_t(ri_ref[s, :], l1w_ref[...]) + l1b_ref[...])
             + temb_ref[...])
        logits = _dot_t(r, l2w_ref[...]) + l2b_ref[...]
        lg0_ref[s, :] = logits[:, 0:1]
        lg1_ref[s, :] = logits[:, 1:2]
        lg2_ref[s, :] = logits[:, 2:3]

        mx = jnp.max(logits, axis=1, keepdims=True)
        mn = jnp.min(logits, axis=1, keepdims=True)
        idx = jax.lax.broadcasted_iota(jnp.int32, logits.shape, 1)
        drop_idx = jnp.max(jnp.where(logits == mn, idx, -1),
                           axis=1, keepdims=True)
        e = jnp.where(idx != drop_idx, jnp.exp(logits - mx), 0.0)
        mw = e / jnp.sum(e, axis=1, keepdims=True)

        out_ref[s, :] = (ft_ref[s, :] * mw[:, 0:1]
                         + fm * mw[:, 1:2]
                         + fi_ref[s, :] * mw[:, 2:3])


_NC, _NS, _L = 2, 16, 16
_NW = _NC * _NS


def _sc_route_kernel(n_tok, lg0_hbm, lg1_hbm, lg2_hbm,
                     mw0_hbm, mw1_hbm, mw2_hbm,
                     in0, in1, in2, out0, out1, out2):
    per_w = n_tok // _NW
    wid = lax.axis_index("s") * _NC + lax.axis_index("c")
    base = wid * per_w
    pltpu.sync_copy(lg0_hbm.at[pl.ds(base, per_w)], in0)
    pltpu.sync_copy(lg1_hbm.at[pl.ds(base, per_w)], in1)
    pltpu.sync_copy(lg2_hbm.at[pl.ds(base, per_w)], in2)
    for c in range(per_w // _L):
        s = pl.ds(c * _L, _L)
        l0, l1, l2 = in0[s], in1[s], in2[s]
        mx = jnp.maximum(jnp.maximum(l0, l1), l2)
        mn = jnp.minimum(jnp.minimum(l0, l1), l2)
        drop = jnp.where(l2 == mn, 2, jnp.where(l1 == mn, 1, 0))
        e0 = jnp.where(drop == 0, 0.0, jnp.exp(l0 - mx))
        e1 = jnp.where(drop == 1, 0.0, jnp.exp(l1 - mx))
        e2 = jnp.where(drop == 2, 0.0, jnp.exp(l2 - mx))
        z = e0 + e1 + e2
        out0[s] = e0 / z
        out1[s] = e1 / z
        out2[s] = e2 / z
    pltpu.sync_copy(out0, mw0_hbm.at[pl.ds(base, per_w)])
    pltpu.sync_copy(out1, mw1_hbm.at[pl.ds(base, per_w)])
    pltpu.sync_copy(out2, mw2_hbm.at[pl.ds(base, per_w)])


@jax.jit
def kernel(F_M, F_T, F_I, router_input, l1_w, l1_b, l2_w, l2_b, task_emb,
           mlp_w1, mlp_b1, mlp_w2, mlp_b2):
    B, T, D = F_T.shape
    D2 = F_M.shape[-1]
    E = l1_w.shape[0]
    N = B * T
    TK = 512

    fm2 = F_M.reshape(N, D2)
    ft2 = F_T.reshape(N, D)
    fi2 = F_I.reshape(N, D)
    ri2 = router_input.reshape(N, D2)
    temb = task_emb[1].reshape(1, E)

    grid = (N // TK,)

    def tok_spec(width):
        return pl.BlockSpec((TK, width), lambda i: (i, 0))

    def const_spec(shape):
        return pl.BlockSpec(shape, lambda i: (0,) * len(shape))

    out, lg0, lg1, lg2 = pl.pallas_call(
        _fused_kernel,
        grid=grid,
        in_specs=[
            tok_spec(D2),
            tok_spec(D),
            tok_spec(D),
            tok_spec(D2),
            const_spec((E, D2)),
            const_spec((1, E)),
            const_spec((3, E)),
            const_spec((1, 3)),
            const_spec((1, E)),
            const_spec((D2, D2)),
            const_spec((1, D2)),
            const_spec((D, D2)),
            const_spec((1, D)),
        ],
        out_specs=[tok_spec(D), tok_spec(1), tok_spec(1), tok_spec(1)],
        out_shape=[
            jax.ShapeDtypeStruct((N, D), jnp.float32),
            jax.ShapeDtypeStruct((N, 1), jnp.float32),
            jax.ShapeDtypeStruct((N, 1), jnp.float32),
            jax.ShapeDtypeStruct((N, 1), jnp.float32),
        ],
        scratch_shapes=[
            pltpu.VMEM((D2, D2), jnp.bfloat16),
            pltpu.VMEM((D, D2), jnp.bfloat16),
        ],
    )(fm2, ft2, fi2, ri2,
      l1_w, l1_b.reshape(1, E), l2_w, l2_b.reshape(1, 3), temb,
      mlp_w1, mlp_b1.reshape(1, D2), mlp_w2, mlp_b2.reshape(1, D))

    per_w = N // _NW
    route = functools.partial(
        pl.kernel,
        mesh=plsc.VectorSubcoreMesh(core_axis_name="c", subcore_axis_name="s"),
        out_type=[
            jax.ShapeDtypeStruct((N,), jnp.float32),
            jax.ShapeDtypeStruct((N,), jnp.float32),
            jax.ShapeDtypeStruct((N,), jnp.float32),
        ],
        scratch_types=[
            pltpu.VMEM((per_w,), jnp.float32),
            pltpu.VMEM((per_w,), jnp.float32),
            pltpu.VMEM((per_w,), jnp.float32),
            pltpu.VMEM((per_w,), jnp.float32),
            pltpu.VMEM((per_w,), jnp.float32),
            pltpu.VMEM((per_w,), jnp.float32),
        ],
    )(functools.partial(_sc_route_kernel, N))
    mw0, mw1, mw2 = route(lg0.reshape(N), lg1.reshape(N), lg2.reshape(N))

    mw = jnp.concatenate(
        [mw0[:, None], mw1[:, None], mw2[:, None]], axis=1)
    return out.reshape(B, T, D), mw.reshape(B, T, 3)


# speedup vs baseline: 1.0998x; 1.0998x over previous
import functools

import jax
import jax.numpy as jnp
from jax import lax
from jax.experimental import pallas as pl
from jax.experimental.pallas import tpu as pltpu
from jax.experimental.pallas import tpu_sc as plsc


def _gelu(x):
    return 0.5 * x * (1.0 + jax.lax.erf(x * 0.7071067811865476))


def _dot_t(x, w):
    return jax.lax.dot_general(
        x, w, dimension_numbers=(((1,), (1,)), ((), ())),
        preferred_element_type=jnp.float32)


_N_SUB = 2


def _fused_kernel(fm_ref, ft_ref, fi_ref, ri_ref,
                  l1w_ref, l1b_ref, l2w_ref, l2b_ref, l2bt_ref, temb_ref,
                  w1_ref, b1_ref, w2_ref, b2_ref,
                  out_ref, lgt_ref, w1b_ref, w2b_ref):
    @pl.when(pl.program_id(0) == 0)
    def _cast_weights():
        w1b_ref[...] = w1_ref[...].astype(jnp.bfloat16)
        w2b_ref[...] = w2_ref[...].astype(jnp.bfloat16)

    sub = out_ref.shape[0] // _N_SUB
    for k in range(_N_SUB):
        s = pl.ds(k * sub, sub)
        x = fm_ref[s, :].astype(jnp.bfloat16)
        h = _gelu(_dot_t(x, w1b_ref[...]) + b1_ref[...])
        fm = _dot_t(h.astype(jnp.bfloat16), w2b_ref[...]) + b2_ref[...]

        r = (_gelu(_dot_t(ri_ref[s, :], l1w_ref[...]) + l1b_ref[...])
             + temb_ref[...])
        logits = _dot_t(r, l2w_ref[...]) + l2b_ref[...]
        lgt_ref[:, s] = (jax.lax.dot_general(
            l2w_ref[...], r, dimension_numbers=(((1,), (1,)), ((), ())),
            preferred_element_type=jnp.float32) + l2bt_ref[...])

        mx = jnp.max(logits, axis=1, keepdims=True)
        mn = jnp.min(logits, axis=1, keepdims=True)
        idx = jax.lax.broadcasted_iota(jnp.int32, logits.shape, 1)
        drop_idx = jnp.max(jnp.where(logits == mn, idx, -1),
                           axis=1, keepdims=True)
        e = jnp.where(idx != drop_idx, jnp.exp(logits - mx), 0.0)
        mw = e / jnp.sum(e, axis=1, keepdims=True)

        out_ref[s, :] = (ft_ref[s, :] * mw[:, 0:1]
                         + fm * mw[:, 1:2]
                         + fi_ref[s, :] * mw[:, 2:3])


_NC, _NS, _L = 2, 16, 16
_NW = _NC * _NS


def _sc_route_kernel(n_tok, lgt_hbm, mwt_hbm, in0, in1, in2,
                     out0, out1, out2):
    per_w = n_tok // _NW
    wid = lax.axis_index("s") * _NC + lax.axis_index("c")
    base = wid * per_w
    pltpu.sync_copy(lgt_hbm.at[pl.ds(base, per_w)], in0)
    pltpu.sync_copy(lgt_hbm.at[pl.ds(n_tok + base, per_w)], in1)
    pltpu.sync_copy(lgt_hbm.at[pl.ds(2 * n_tok + base, per_w)], in2)
    for c in range(per_w // _L):
        s = pl.ds(c * _L, _L)
        l0, l1, l2 = in0[s], in1[s], in2[s]
        mx = jnp.maximum(jnp.maximum(l0, l1), l2)
        mn = jnp.minimum(jnp.minimum(l0, l1), l2)
        drop = jnp.where(l2 == mn, 2, jnp.where(l1 == mn, 1, 0))
        e0 = jnp.where(drop == 0, 0.0, jnp.exp(l0 - mx))
        e1 = jnp.where(drop == 1, 0.0, jnp.exp(l1 - mx))
        e2 = jnp.where(drop == 2, 0.0, jnp.exp(l2 - mx))
        z = e0 + e1 + e2
        out0[s] = e0 / z
        out1[s] = e1 / z
        out2[s] = e2 / z
    pltpu.sync_copy(out0, mwt_hbm.at[pl.ds(base, per_w)])
    pltpu.sync_copy(out1, mwt_hbm.at[pl.ds(n_tok + base, per_w)])
    pltpu.sync_copy(out2, mwt_hbm.at[pl.ds(2 * n_tok + base, per_w)])


@jax.jit
def kernel(F_M, F_T, F_I, router_input, l1_w, l1_b, l2_w, l2_b, task_emb,
           mlp_w1, mlp_b1, mlp_w2, mlp_b2):
    B, T, D = F_T.shape
    D2 = F_M.shape[-1]
    E = l1_w.shape[0]
    N = B * T
    TK = 512

    fm2 = F_M.reshape(N, D2)
    ft2 = F_T.reshape(N, D)
    fi2 = F_I.reshape(N, D)
    ri2 = router_input.reshape(N, D2)
    temb = task_emb[1].reshape(1, E)

    grid = (N // TK,)

    def tok_spec(width):
        return pl.BlockSpec((TK, width), lambda i: (i, 0))

    def const_spec(shape):
        return pl.BlockSpec(shape, lambda i: (0,) * len(shape))

    out, lgt = pl.pallas_call(
        _fused_kernel,
        grid=grid,
        in_specs=[
            tok_spec(D2),
            tok_spec(D),
            tok_spec(D),
            tok_spec(D2),
            const_spec((E, D2)),
            const_spec((1, E)),
            const_spec((3, E)),
            const_spec((1, 3)),
            const_spec((3, 1)),
            const_spec((1, E)),
            const_spec((D2, D2)),
            const_spec((1, D2)),
            const_spec((D, D2)),
            const_spec((1, D)),
        ],
        out_specs=[tok_spec(D), pl.BlockSpec((3, TK), lambda i: (0, i))],
        out_shape=[
            jax.ShapeDtypeStruct((N, D), jnp.float32),
            jax.ShapeDtypeStruct((3, N), jnp.float32),
        ],
        scratch_shapes=[
            pltpu.VMEM((D2, D2), jnp.bfloat16),
            pltpu.VMEM((D, D2), jnp.bfloat16),
        ],
    )(fm2, ft2, fi2, ri2,
      l1_w, l1_b.reshape(1, E), l2_w, l2_b.reshape(1, 3),
      l2_b.reshape(3, 1), temb,
      mlp_w1, mlp_b1.reshape(1, D2), mlp_w2, mlp_b2.reshape(1, D))

    per_w = N // _NW
    route = functools.partial(
        pl.kernel,
        mesh=plsc.VectorSubcoreMesh(core_axis_name="c", subcore_axis_name="s"),
        out_type=jax.ShapeDtypeStruct((3 * N,), jnp.float32),
        scratch_types=[
            pltpu.VMEM((per_w,), jnp.float32),
            pltpu.VMEM((per_w,), jnp.float32),
            pltpu.VMEM((per_w,), jnp.float32),
            pltpu.VMEM((per_w,), jnp.float32),
            pltpu.VMEM((per_w,), jnp.float32),
            pltpu.VMEM((per_w,), jnp.float32),
        ],
    )(functools.partial(_sc_route_kernel, N))
    mwt = route(lgt.reshape(3 * N))

    mw = mwt.reshape(3, N).T
    return out.reshape(B, T, D), mw.reshape(B, T, 3)


# speedup vs baseline: 1.1127x; 1.0117x over previous
import functools

import jax
import jax.numpy as jnp
from jax import lax
from jax.experimental import pallas as pl
from jax.experimental.pallas import tpu as pltpu
from jax.experimental.pallas import tpu_sc as plsc


def _gelu(x):
    return 0.5 * x * (1.0 + jax.lax.erf(x * 0.7071067811865476))


def _dot_t(x, w):
    return jax.lax.dot_general(
        x, w, dimension_numbers=(((1,), (1,)), ((), ())),
        preferred_element_type=jnp.float32)


_N_SUB = 2


def _router_kernel(ri_ref, l1w_ref, l1b_ref, l2w_ref, l2bt_ref, temb_ref,
                   lgt_ref):
    tk = ri_ref.shape[0]
    sub = tk // _N_SUB
    for k in range(_N_SUB):
        s = pl.ds(k * sub, sub)
        r = (_gelu(_dot_t(ri_ref[s, :], l1w_ref[...]) + l1b_ref[...])
             + temb_ref[...])
        lgt_ref[:, s] = (jax.lax.dot_general(
            l2w_ref[...], r, dimension_numbers=(((1,), (1,)), ((), ())),
            preferred_element_type=jnp.float32) + l2bt_ref[...])


def _mlp_fuse_kernel(fm_ref, ft_ref, fi_ref, lgt_ref,
                     w1_ref, b1_ref, w2_ref, b2_ref,
                     out_ref, w1b_ref, w2b_ref):
    @pl.when(pl.program_id(0) == 0)
    def _cast_weights():
        w1b_ref[...] = w1_ref[...].astype(jnp.bfloat16)
        w2b_ref[...] = w2_ref[...].astype(jnp.bfloat16)

    sub = out_ref.shape[0] // _N_SUB
    for k in range(_N_SUB):
        s = pl.ds(k * sub, sub)
        x = fm_ref[s, :].astype(jnp.bfloat16)
        h = _gelu(_dot_t(x, w1b_ref[...]) + b1_ref[...])
        fm = _dot_t(h.astype(jnp.bfloat16), w2b_ref[...]) + b2_ref[...]

        logits = jnp.transpose(lgt_ref[:, s])
        mx = jnp.max(logits, axis=1, keepdims=True)
        mn = jnp.min(logits, axis=1, keepdims=True)
        idx = jax.lax.broadcasted_iota(jnp.int32, logits.shape, 1)
        drop_idx = jnp.max(jnp.where(logits == mn, idx, -1),
                           axis=1, keepdims=True)
        e = jnp.where(idx != drop_idx, jnp.exp(logits - mx), 0.0)
        mw = e / jnp.sum(e, axis=1, keepdims=True)

        out_ref[s, :] = (ft_ref[s, :] * mw[:, 0:1]
                         + fm * mw[:, 1:2]
                         + fi_ref[s, :] * mw[:, 2:3])


_NC, _NS, _L = 2, 16, 16
_NW = _NC * _NS


def _sc_route_kernel(n_tok, lgt_hbm, mwt_hbm, in0, in1, in2,
                     out0, out1, out2):
    per_w = n_tok // _NW
    wid = lax.axis_index("s") * _NC + lax.axis_index("c")
    base = wid * per_w
    pltpu.sync_copy(lgt_hbm.at[pl.ds(base, per_w)], in0)
    pltpu.sync_copy(lgt_hbm.at[pl.ds(n_tok + base, per_w)], in1)
    pltpu.sync_copy(lgt_hbm.at[pl.ds(2 * n_tok + base, per_w)], in2)
    for c in range(per_w // _L):
        s = pl.ds(c * _L, _L)
        l0, l1, l2 = in0[s], in1[s], in2[s]
        mx = jnp.maximum(jnp.maximum(l0, l1), l2)
        mn = jnp.minimum(jnp.minimum(l0, l1), l2)
        drop = jnp.where(l2 == mn, 2, jnp.where(l1 == mn, 1, 0))
        e0 = jnp.where(drop == 0, 0.0, jnp.exp(l0 - mx))
        e1 = jnp.where(drop == 1, 0.0, jnp.exp(l1 - mx))
        e2 = jnp.where(drop == 2, 0.0, jnp.exp(l2 - mx))
        z = e0 + e1 + e2
        out0[s] = e0 / z
        out1[s] = e1 / z
        out2[s] = e2 / z
    pltpu.sync_copy(out0, mwt_hbm.at[pl.ds(base, per_w)])
    pltpu.sync_copy(out1, mwt_hbm.at[pl.ds(n_tok + base, per_w)])
    pltpu.sync_copy(out2, mwt_hbm.at[pl.ds(2 * n_tok + base, per_w)])


@jax.jit
def kernel(F_M, F_T, F_I, router_input, l1_w, l1_b, l2_w, l2_b, task_emb,
           mlp_w1, mlp_b1, mlp_w2, mlp_b2):
    B, T, D = F_T.shape
    D2 = F_M.shape[-1]
    E = l1_w.shape[0]
    N = B * T
    TKA = 2048
    TK = 512

    fm2 = F_M.reshape(N, D2)
    ft2 = F_T.reshape(N, D)
    fi2 = F_I.reshape(N, D)
    ri2 = router_input.reshape(N, D2)
    temb = task_emb[1].reshape(1, E)

    def tok_spec(tk, width):
        return pl.BlockSpec((tk, width), lambda i: (i, 0))

    def const_spec(shape):
        return pl.BlockSpec(shape, lambda i: (0,) * len(shape))

    lgt = pl.pallas_call(
        _router_kernel,
        grid=(N // TKA,),
        in_specs=[
            tok_spec(TKA, D2),
            const_spec((E, D2)),
            const_spec((1, E)),
            const_spec((3, E)),
            const_spec((3, 1)),
            const_spec((1, E)),
        ],
        out_specs=pl.BlockSpec((3, TKA), lambda i: (0, i)),
        out_shape=jax.ShapeDtypeStruct((3, N), jnp.float32),
    )(ri2, l1_w, l1_b.reshape(1, E), l2_w, l2_b.reshape(3, 1), temb)

    per_w = N // _NW
    route = functools.partial(
        pl.kernel,
        mesh=plsc.VectorSubcoreMesh(core_axis_name="c", subcore_axis_name="s"),
        out_type=jax.ShapeDtypeStruct((3 * N,), jnp.float32),
        scratch_types=[
            pltpu.VMEM((per_w,), jnp.float32),
            pltpu.VMEM((per_w,), jnp.float32),
            pltpu.VMEM((per_w,), jnp.float32),
            pltpu.VMEM((per_w,), jnp.float32),
            pltpu.VMEM((per_w,), jnp.float32),
            pltpu.VMEM((per_w,), jnp.float32),
        ],
    )(functools.partial(_sc_route_kernel, N))
    mwt = route(lgt.reshape(3 * N))

    out = pl.pallas_call(
        _mlp_fuse_kernel,
        grid=(N // TK,),
        in_specs=[
            tok_spec(TK, D2),
            tok_spec(TK, D),
            tok_spec(TK, D),
            pl.BlockSpec((3, TK), lambda i: (0, i)),
            const_spec((D2, D2)),
            const_spec((1, D2)),
            const_spec((D, D2)),
            const_spec((1, D)),
        ],
        out_specs=tok_spec(TK, D),
        out_shape=jax.ShapeDtypeStruct((N, D), jnp.float32),
        scratch_shapes=[
            pltpu.VMEM((D2, D2), jnp.bfloat16),
            pltpu.VMEM((D, D2), jnp.bfloat16),
        ],
    )(fm2, ft2, fi2, lgt,
      mlp_w1, mlp_b1.reshape(1, D2), mlp_w2, mlp_b2.reshape(1, D))

    mw = mwt.reshape(3, N).T
    return out.reshape(B, T, D), mw.reshape(B, T, 3)


# speedup vs baseline: 1.1698x; 1.0513x over previous
import functools

import jax
import jax.numpy as jnp
from jax import lax
from jax.experimental import pallas as pl
from jax.experimental.pallas import tpu as pltpu
from jax.experimental.pallas import tpu_sc as plsc


def _gelu(x):
    return 0.5 * x * (1.0 + jax.lax.erf(x * 0.7071067811865476))


def _dot_t(x, w):
    return jax.lax.dot_general(
        x, w, dimension_numbers=(((1,), (1,)), ((), ())),
        preferred_element_type=jnp.float32)


_N_SUB = 2


def _router_kernel(ri_ref, l1w_ref, l1b_ref, l2w_ref, l2bt_ref, temb_ref,
                   lgt_ref):
    tk = ri_ref.shape[0]
    sub = tk // _N_SUB
    for k in range(_N_SUB):
        s = pl.ds(k * sub, sub)
        r = (_gelu(_dot_t(ri_ref[s, :], l1w_ref[...]) + l1b_ref[...])
             + temb_ref[...])
        lgt_ref[:, s] = (jax.lax.dot_general(
            l2w_ref[...], r, dimension_numbers=(((1,), (1,)), ((), ())),
            preferred_element_type=jnp.float32) + l2bt_ref[...])


def _mlp_fuse_kernel(fm_ref, ft_ref, fi_ref, lgt_ref,
                     w1_ref, b1_ref, w2_ref, b2_ref,
                     out_ref, w1b_ref, w2b_ref):
    @pl.when(pl.program_id(0) == 0)
    def _cast_weights():
        w1b_ref[...] = w1_ref[...].astype(jnp.bfloat16)
        w2b_ref[...] = w2_ref[...].astype(jnp.bfloat16)

    sub = out_ref.shape[0] // _N_SUB
    for k in range(_N_SUB):
        s = pl.ds(k * sub, sub)
        x = fm_ref[s, :].astype(jnp.bfloat16)
        h = _gelu(_dot_t(x, w1b_ref[...]) + b1_ref[...])
        fm = _dot_t(h.astype(jnp.bfloat16), w2b_ref[...]) + b2_ref[...]

        logits = jnp.transpose(lgt_ref[:, s])
        mx = jnp.max(logits, axis=1, keepdims=True)
        mn = jnp.min(logits, axis=1, keepdims=True)
        idx = jax.lax.broadcasted_iota(jnp.int32, logits.shape, 1)
        drop_idx = jnp.max(jnp.where(logits == mn, idx, -1),
                           axis=1, keepdims=True)
        e = jnp.where(idx != drop_idx, jnp.exp(logits - mx), 0.0)
        mw = e / jnp.sum(e, axis=1, keepdims=True)

        out_ref[s, :] = (ft_ref[s, :] * mw[:, 0:1]
                         + fm * mw[:, 1:2]
                         + fi_ref[s, :] * mw[:, 2:3])


_NC, _NS, _L = 2, 16, 16
_NW = _NC * _NS


def _sc_route_kernel(n_tok, lgt_hbm, mwt_hbm, in0, in1, in2,
                     out0, out1, out2):
    per_w = n_tok // _NW
    wid = lax.axis_index("s") * _NC + lax.axis_index("c")
    base = wid * per_w
    pltpu.sync_copy(lgt_hbm.at[pl.ds(base, per_w)], in0)
    pltpu.sync_copy(lgt_hbm.at[pl.ds(n_tok + base, per_w)], in1)
    pltpu.sync_copy(lgt_hbm.at[pl.ds(2 * n_tok + base, per_w)], in2)
    for c in range(per_w // _L):
        s = pl.ds(c * _L, _L)
        l0, l1, l2 = in0[s], in1[s], in2[s]
        mx = jnp.maximum(jnp.maximum(l0, l1), l2)
        mn = jnp.minimum(jnp.minimum(l0, l1), l2)
        drop = jnp.where(l2 == mn, 2, jnp.where(l1 == mn, 1, 0))
        e0 = jnp.where(drop == 0, 0.0, jnp.exp(l0 - mx))
        e1 = jnp.where(drop == 1, 0.0, jnp.exp(l1 - mx))
        e2 = jnp.where(drop == 2, 0.0, jnp.exp(l2 - mx))
        z = e0 + e1 + e2
        out0[s] = e0 / z
        out1[s] = e1 / z
        out2[s] = e2 / z
    pltpu.sync_copy(out0, mwt_hbm.at[pl.ds(base, per_w)])
    pltpu.sync_copy(out1, mwt_hbm.at[pl.ds(n_tok + base, per_w)])
    pltpu.sync_copy(out2, mwt_hbm.at[pl.ds(2 * n_tok + base, per_w)])


@jax.jit
def kernel(F_M, F_T, F_I, router_input, l1_w, l1_b, l2_w, l2_b, task_emb,
           mlp_w1, mlp_b1, mlp_w2, mlp_b2):
    B, T, D = F_T.shape
    D2 = F_M.shape[-1]
    E = l1_w.shape[0]
    N = B * T
    TKA = 2048
    TK = 1024

    fm2 = F_M.reshape(N, D2)
    ft2 = F_T.reshape(N, D)
    fi2 = F_I.reshape(N, D)
    ri2 = router_input.reshape(N, D2)
    temb = task_emb[1].reshape(1, E)

    def tok_spec(tk, width):
        return pl.BlockSpec((tk, width), lambda i: (i, 0))

    def const_spec(shape):
        return pl.BlockSpec(shape, lambda i: (0,) * len(shape))

    lgt = pl.pallas_call(
        _router_kernel,
        grid=(N // TKA,),
        in_specs=[
            tok_spec(TKA, D2),
            const_spec((E, D2)),
            const_spec((1, E)),
            const_spec((3, E)),
            const_spec((3, 1)),
            const_spec((1, E)),
        ],
        out_specs=pl.BlockSpec((3, TKA), lambda i: (0, i)),
        out_shape=jax.ShapeDtypeStruct((3, N), jnp.float32),
    )(ri2, l1_w, l1_b.reshape(1, E), l2_w, l2_b.reshape(3, 1), temb)

    per_w = N // _NW
    route = functools.partial(
        pl.kernel,
        mesh=plsc.VectorSubcoreMesh(core_axis_name="c", subcore_axis_name="s"),
        out_type=jax.ShapeDtypeStruct((3 * N,), jnp.float32),
        scratch_types=[
            pltpu.VMEM((per_w,), jnp.float32),
            pltpu.VMEM((per_w,), jnp.float32),
            pltpu.VMEM((per_w,), jnp.float32),
            pltpu.VMEM((per_w,), jnp.float32),
            pltpu.VMEM((per_w,), jnp.float32),
            pltpu.VMEM((per_w,), jnp.float32),
        ],
    )(functools.partial(_sc_route_kernel, N))
    mwt = route(lgt.reshape(3 * N))

    out = pl.pallas_call(
        _mlp_fuse_kernel,
        grid=(N // TK,),
        in_specs=[
            tok_spec(TK, D2),
            tok_spec(TK, D),
            tok_spec(TK, D),
            pl.BlockSpec((3, TK), lambda i: (0, i)),
            const_spec((D2, D2)),
            const_spec((1, D2)),
            const_spec((D, D2)),
            const_spec((1, D)),
        ],
        out_specs=tok_spec(TK, D),
        out_shape=jax.ShapeDtypeStruct((N, D), jnp.float32),
        scratch_shapes=[
            pltpu.VMEM((D2, D2), jnp.bfloat16),
            pltpu.VMEM((D, D2), jnp.bfloat16),
        ],
    )(fm2, ft2, fi2, lgt,
      mlp_w1, mlp_b1.reshape(1, D2), mlp_w2, mlp_b2.reshape(1, D))

    mw = mwt.reshape(3, N).T
    return out.reshape(B, T, D), mw.reshape(B, T, 3)
